# baseline (speedup 1.0000x reference)
import jax
import jax.numpy as jnp
from jax import lax
from jax.experimental import pallas as pl
from jax.experimental.pallas import tpu as pltpu
from jax.experimental.pallas import tpu_sc as plsc

_S = 200
_D = 128
_ROWS_PER_WORKER = 8
_NUM_WORKERS = _S // _ROWS_PER_WORKER
_BB = 128
_K_PRE = 1


def _sc_combine_body(tt_hbm, st_hbm, lt_hbm, ti_hbm, si_hbm, li_hbm,
                     c_hbm, ti_v, si_v, li_v, rt_v, rs_v, rl_v, out_v,
                     sem):
    wid = lax.axis_index("s") * 2 + lax.axis_index("c")

    @pl.when(wid < _NUM_WORKERS)
    def _():
        base = wid * _ROWS_PER_WORKER
        rows = pl.ds(base, _ROWS_PER_WORKER)
        cp1 = pltpu.async_copy(ti_hbm.at[rows], ti_v, sem)
        cp2 = pltpu.async_copy(si_hbm.at[rows], si_v, sem)
        cp3 = pltpu.async_copy(li_hbm.at[rows], li_v, sem)
        cp1.wait(); cp2.wait(); cp3.wait()
        g1 = pltpu.async_copy(tt_hbm.at[ti_v], rt_v, sem)
        g2 = pltpu.async_copy(st_hbm.at[si_v], rs_v, sem)
        g3 = pltpu.async_copy(lt_hbm.at[li_v], rl_v, sem)
        g1.wait(); g2.wait(); g3.wait()
        for r in range(_ROWS_PER_WORKER):
            for k in range(_D // 16):
                sl = pl.ds(k * 16, 16)
                out_v[r, sl] = rt_v[r, sl] + rs_v[r, sl] + rl_v[r, sl]
        pltpu.sync_copy(out_v, c_hbm.at[rows])


def _make_sc_combine():
    return pl.kernel(
        _sc_combine_body,
        out_type=jax.ShapeDtypeStruct((_S, _D), jnp.float32),
        mesh=plsc.VectorSubcoreMesh(
            core_axis_name="c", subcore_axis_name="s",
            num_cores=2, num_subcores=16),
        scratch_types=[
            pltpu.VMEM((_ROWS_PER_WORKER,), jnp.int32),
            pltpu.VMEM((_ROWS_PER_WORKER,), jnp.int32),
            pltpu.VMEM((_ROWS_PER_WORKER,), jnp.int32),
            pltpu.VMEM((_ROWS_PER_WORKER, _D), jnp.float32),
            pltpu.VMEM((_ROWS_PER_WORKER, _D), jnp.float32),
            pltpu.VMEM((_ROWS_PER_WORKER, _D), jnp.float32),
            pltpu.VMEM((_ROWS_PER_WORKER, _D), jnp.float32),
            pltpu.SemaphoreType.DMA,
        ],
    )


def _tc_combine_body(ti_ref, si_ref, li_ref, tt_ref, st_ref, lt_ref, c_ref):
    S = ti_ref.shape[1]

    def emb(ids_ref, table_ref):
        n = table_ref.shape[0]
        iota = jax.lax.broadcasted_iota(jnp.int32, (n, S), 0)
        oh_t = (ids_ref[...] == iota).astype(jnp.float32)
        return jax.lax.dot_general(
            oh_t, table_ref[...],
            dimension_numbers=(((0,), (0,)), ((), ())),
            preferred_element_type=jnp.float32)

    c_ref[...] = emb(ti_ref, tt_ref) + emb(si_ref, st_ref) + emb(li_ref, lt_ref)


def _fuse_body(tvT_ref, w_ref, b_ref, c_ref, out_ref):
    S, BB = tvT_ref.shape
    tv = tvT_ref[...].T.reshape(BB, S, 1)
    w = w_ref[...][None, None, :]
    cb = (c_ref[...] + b_ref[...][None, :])[None]
    out_ref[...] = tv * w + cb


def _fuse_tail_body(dst_ref, tv_ref, w_ref, b_ref, c_ref, out_ref):
    del dst_ref
    _fuse_body(tv_ref, w_ref, b_ref, c_ref, out_ref)


def kernel(token_values, W_val, b_val, type_table, side_table, slot_table,
           token_type_ids, token_side_ids, token_slot_ids):
    B, S = token_values.shape
    D = W_val.shape[0]

    w_vec = W_val.reshape(D)
    tvT = token_values.T
    ti = token_type_ids.reshape(1, S)
    si = token_side_ids.reshape(1, S)
    li = token_slot_ids.reshape(1, S)

    c_sc = _make_sc_combine()(type_table, side_table, slot_table,
                              token_type_ids, token_side_ids,
                              token_slot_ids)

    c_tc = pl.pallas_call(
        _tc_combine_body,
        out_shape=jax.ShapeDtypeStruct((S, D), jnp.float32),
    )(ti, si, li, type_table, side_table, slot_table)

    fuse_specs = dict(
        out_shape=jax.ShapeDtypeStruct((B, S, D), jnp.float32),
        compiler_params=pltpu.CompilerParams(
            dimension_semantics=("parallel",)),
    )

    head = pl.pallas_call(
        _fuse_body,
        grid=(_K_PRE,),
        in_specs=[
            pl.BlockSpec((S, _BB), lambda i: (0, i)),
            pl.BlockSpec((D,), lambda i: (0,)),
            pl.BlockSpec((D,), lambda i: (0,)),
            pl.BlockSpec((S, D), lambda i: (0, 0)),
        ],
        out_specs=pl.BlockSpec((_BB, S, D), lambda i: (i, 0, 0)),
        **fuse_specs,
    )(tvT, w_vec, b_val, c_tc)

    out = pl.pallas_call(
        _fuse_tail_body,
        grid=(B // _BB - _K_PRE,),
        in_specs=[
            pl.BlockSpec(memory_space=pltpu.MemorySpace.HBM),
            pl.BlockSpec((S, _BB), lambda i: (0, i + _K_PRE)),
            pl.BlockSpec((D,), lambda i: (0,)),
            pl.BlockSpec((D,), lambda i: (0,)),
            pl.BlockSpec((S, D), lambda i: (0, 0)),
        ],
        out_specs=pl.BlockSpec((_BB, S, D), lambda i: (i + _K_PRE, 0, 0)),
        input_output_aliases={0: 0},
        **fuse_specs,
    )(head, tvT, w_vec, b_val, c_sc)

    return out


# speedup vs baseline: 1.0009x; 1.0009x over previous
import jax
import jax.numpy as jnp
from jax import lax
from jax.experimental import pallas as pl
from jax.experimental.pallas import tpu as pltpu
from jax.experimental.pallas import tpu_sc as plsc

_S = 200
_D = 128
_ROWS_PER_WORKER = 8
_NUM_WORKERS = _S // _ROWS_PER_WORKER
_BB = 128
_K_PRE = 2


def _sc_combine_body(tt_hbm, st_hbm, lt_hbm, ti_hbm, si_hbm, li_hbm,
                     c_hbm, ti_v, si_v, li_v, rt_v, rs_v, rl_v, out_v,
                     sem):
    wid = lax.axis_index("s") * 2 + lax.axis_index("c")

    @pl.when(wid < _NUM_WORKERS)
    def _():
        base = wid * _ROWS_PER_WORKER
        rows = pl.ds(base, _ROWS_PER_WORKER)
        cp1 = pltpu.async_copy(ti_hbm.at[rows], ti_v, sem)
        cp2 = pltpu.async_copy(si_hbm.at[rows], si_v, sem)
        cp3 = pltpu.async_copy(li_hbm.at[rows], li_v, sem)
        cp1.wait(); cp2.wait(); cp3.wait()
        g1 = pltpu.async_copy(tt_hbm.at[ti_v], rt_v, sem)
        g2 = pltpu.async_copy(st_hbm.at[si_v], rs_v, sem)
        g3 = pltpu.async_copy(lt_hbm.at[li_v], rl_v, sem)
        g1.wait(); g2.wait(); g3.wait()
        for r in range(_ROWS_PER_WORKER):
            for k in range(_D // 16):
                sl = pl.ds(k * 16, 16)
                out_v[r, sl] = rt_v[r, sl] + rs_v[r, sl] + rl_v[r, sl]
        pltpu.sync_copy(out_v, c_hbm.at[rows])


def _make_sc_combine():
    return pl.kernel(
        _sc_combine_body,
        out_type=jax.ShapeDtypeStruct((_S, _D), jnp.float32),
        mesh=plsc.VectorSubcoreMesh(
            core_axis_name="c", subcore_axis_name="s",
            num_cores=2, num_subcores=16),
        scratch_types=[
            pltpu.VMEM((_ROWS_PER_WORKER,), jnp.int32),
            pltpu.VMEM((_ROWS_PER_WORKER,), jnp.int32),
            pltpu.VMEM((_ROWS_PER_WORKER,), jnp.int32),
            pltpu.VMEM((_ROWS_PER_WORKER, _D), jnp.float32),
            pltpu.VMEM((_ROWS_PER_WORKER, _D), jnp.float32),
            pltpu.VMEM((_ROWS_PER_WORKER, _D), jnp.float32),
            pltpu.VMEM((_ROWS_PER_WORKER, _D), jnp.float32),
            pltpu.SemaphoreType.DMA,
        ],
    )


def _tc_combine_body(ti_ref, si_ref, li_ref, tt_ref, st_ref, lt_ref, c_ref):
    S = ti_ref.shape[1]

    def emb(ids_ref, table_ref):
        n = table_ref.shape[0]
        iota = jax.lax.broadcasted_iota(jnp.int32, (n, S), 0)
        oh_t = (ids_ref[...] == iota).astype(jnp.float32)
        return jax.lax.dot_general(
            oh_t, table_ref[...],
            dimension_numbers=(((0,), (0,)), ((), ())),
            preferred_element_type=jnp.float32)

    c_ref[...] = emb(ti_ref, tt_ref) + emb(si_ref, st_ref) + emb(li_ref, lt_ref)


def _fuse_body(tvT_ref, w_ref, b_ref, c_ref, out_ref):
    S, BB = tvT_ref.shape
    tv = tvT_ref[...].T.reshape(BB, S, 1)
    w = w_ref[...][None, None, :]
    cb = (c_ref[...] + b_ref[...][None, :])[None]
    out_ref[...] = tv * w + cb


def _fuse_tail_body(dst_ref, tv_ref, w_ref, b_ref, c_ref, out_ref):
    del dst_ref
    _fuse_body(tv_ref, w_ref, b_ref, c_ref, out_ref)


def kernel(token_values, W_val, b_val, type_table, side_table, slot_table,
           token_type_ids, token_side_ids, token_slot_ids):
    B, S = token_values.shape
    D = W_val.shape[0]

    w_vec = W_val.reshape(D)
    tvT = token_values.T
    ti = token_type_ids.reshape(1, S)
    si = token_side_ids.reshape(1, S)
    li = token_slot_ids.reshape(1, S)

    c_sc = _make_sc_combine()(type_table, side_table, slot_table,
                              token_type_ids, token_side_ids,
                              token_slot_ids)

    c_tc = pl.pallas_call(
        _tc_combine_body,
        out_shape=jax.ShapeDtypeStruct((S, D), jnp.float32),
    )(ti, si, li, type_table, side_table, slot_table)

    fuse_specs = dict(
        out_shape=jax.ShapeDtypeStruct((B, S, D), jnp.float32),
        compiler_params=pltpu.CompilerParams(
            dimension_semantics=("parallel",)),
    )

    head = pl.pallas_call(
        _fuse_body,
        grid=(_K_PRE,),
        in_specs=[
            pl.BlockSpec((S, _BB), lambda i: (0, i)),
            pl.BlockSpec((D,), lambda i: (0,)),
            pl.BlockSpec((D,), lambda i: (0,)),
            pl.BlockSpec((S, D), lambda i: (0, 0)),
        ],
        out_specs=pl.BlockSpec((_BB, S, D), lambda i: (i, 0, 0)),
        **fuse_specs,
    )(tvT, w_vec, b_val, c_tc)

    out = pl.pallas_call(
        _fuse_tail_body,
        grid=(B // _BB - _K_PRE,),
        in_specs=[
            pl.BlockSpec(memory_space=pltpu.MemorySpace.HBM),
            pl.BlockSpec((S, _BB), lambda i: (0, i + _K_PRE)),
            pl.BlockSpec((D,), lambda i: (0,)),
            pl.BlockSpec((D,), lambda i: (0,)),
            pl.BlockSpec((S, D), lambda i: (0, 0)),
        ],
        out_specs=pl.BlockSpec((_BB, S, D), lambda i: (i + _K_PRE, 0, 0)),
        input_output_aliases={0: 0},
        **fuse_specs,
    )(head, tvT, w_vec, b_val, c_sc)

    return out
